# baseline (speedup 1.0000x reference)
import functools

import jax
import jax.numpy as jnp
from jax import lax
from jax.experimental import pallas as pl
from jax.experimental.pallas import tpu as pltpu
from jax.experimental.pallas import tpu_sc as plsc

NUM_CLASSES = 1000
TOTAL_ROWS = 4096 * 20
NW = 32
ROWS_PER_W = TOTAL_ROWS // NW
ROWS_PER_BATCH = 160
GROUPS = ROWS_PER_BATCH // 16
BATCHES = ROWS_PER_W // ROWS_PER_BATCH
BATCH_BYTES = ROWS_PER_BATCH * NUM_CLASSES
PAD = 64

_mesh = plsc.VectorSubcoreMesh(core_axis_name="c", subcore_axis_name="s")


@functools.partial(
    pl.kernel,
    mesh=_mesh,
    compiler_params=pltpu.CompilerParams(use_tc_tiling_on_sc=False),
    out_type=jax.ShapeDtypeStruct((TOTAL_ROWS * NUM_CLASSES,), jnp.int8),
    scratch_types=[
        pltpu.VMEM((ROWS_PER_W,), jnp.int32),
        pltpu.VMEM((BATCH_BYTES + PAD,), jnp.int8),
        pltpu.VMEM((65 * 64,), jnp.int8),
    ],
)
def _sc_onehot(cls_hbm, zeros_hbm, table_hbm, out_hbm, cls_v, buf, table_v):
    wid = lax.axis_index("s") * 2 + lax.axis_index("c")
    row0 = wid * ROWS_PER_W
    pltpu.sync_copy(cls_hbm.at[pl.ds(row0, ROWS_PER_W)], cls_v)
    pltpu.sync_copy(zeros_hbm, buf)
    pltpu.sync_copy(table_hbm, table_v)

    def scatter_batch(i, table_row_of):
        for g in range(GROUPS):
            r0 = i * ROWS_PER_BATCH + g * 16
            cls16 = cls_v[pl.ds(r0, 16)]
            for k in range(16):
                r = g * 16 + k
                c = cls16[k]
                flat = r * NUM_CLASSES + c
                cb = pl.multiple_of(flat & ~63, 64)
                toff = pl.multiple_of(table_row_of(flat - cb), 64)
                buf[pl.ds(cb, 64)] = table_v[pl.ds(toff, 64)]

    def batch(i, carry):
        scatter_batch(i, lambda t: t * 64)
        pltpu.sync_copy(
            buf.at[pl.ds(0, BATCH_BYTES)],
            out_hbm.at[pl.ds((row0 + i * ROWS_PER_BATCH) * NUM_CLASSES,
                             BATCH_BYTES)])
        scatter_batch(i, lambda t: jnp.int32(64 * 64))
        return carry

    lax.fori_loop(0, BATCHES, batch, 0)


def kernel(classes):
    cls_flat = classes.reshape(-1)
    zeros_help = jnp.zeros((BATCH_BYTES + PAD,), jnp.int8)
    eye = jnp.eye(64, dtype=jnp.int8)
    table = jnp.concatenate(
        [eye, jnp.zeros((1, 64), jnp.int8)]).reshape(-1)
    flat8 = _sc_onehot(cls_flat, zeros_help, table)
    return flat8.reshape(4096, 20, NUM_CLASSES).astype(jnp.int32)


# speedup vs baseline: 1.7215x; 1.7215x over previous
import functools

import jax
import jax.numpy as jnp
from jax import lax
from jax.experimental import pallas as pl
from jax.experimental.pallas import tpu as pltpu
from jax.experimental.pallas import tpu_sc as plsc

NUM_CLASSES = 1000
TOTAL_ROWS = 4096 * 20
NW = 32
ROWS_PER_W = TOTAL_ROWS // NW
ROWS_PER_BATCH = 80
GROUPS = ROWS_PER_BATCH // 16
BATCHES = ROWS_PER_W // ROWS_PER_BATCH
BATCH_WORDS = ROWS_PER_BATCH * NUM_CLASSES

_mesh = plsc.VectorSubcoreMesh(core_axis_name="c", subcore_axis_name="s")


@functools.partial(
    pl.kernel,
    mesh=_mesh,
    compiler_params=pltpu.CompilerParams(use_tc_tiling_on_sc=False),
    out_type=jax.ShapeDtypeStruct((TOTAL_ROWS * NUM_CLASSES,), jnp.int32),
    scratch_types=[
        pltpu.VMEM((ROWS_PER_W,), jnp.int32),
        pltpu.VMEM((BATCH_WORDS,), jnp.int32),
        pltpu.VMEM((2 * 16 * 16,), jnp.int32),
    ],
)
def _sc_onehot(cls_hbm, zeros_hbm, table_hbm, out_hbm, cls_v, buf, table_v):
    wid = lax.axis_index("s") * 2 + lax.axis_index("c")
    row0 = wid * ROWS_PER_W
    pltpu.sync_copy(cls_hbm.at[pl.ds(row0, ROWS_PER_W)], cls_v)
    pltpu.sync_copy(zeros_hbm, buf)
    pltpu.sync_copy(table_hbm, table_v)

    def scatter_batch(i, table_base, combine):
        for g in range(GROUPS):
            r0 = i * ROWS_PER_BATCH + g * 16
            cls16 = cls_v[pl.ds(r0, 16)]
            for k in range(16):
                r = g * 16 + k
                c = cls16[k]
                flat = r * NUM_CLASSES + c
                cb = pl.multiple_of(flat & ~15, 16)
                toff = pl.multiple_of(table_base + (flat - cb) * 16, 16)
                win = table_v[pl.ds(toff, 16)]
                buf[pl.ds(cb, 16)] = combine(buf[pl.ds(cb, 16)], win)

    def batch(i, carry):
        scatter_batch(i, 0, jnp.bitwise_or)
        pltpu.sync_copy(
            buf,
            out_hbm.at[pl.ds((row0 + i * ROWS_PER_BATCH) * NUM_CLASSES,
                             BATCH_WORDS)])
        scatter_batch(i, 16 * 16, jnp.bitwise_and)
        return carry

    lax.fori_loop(0, BATCHES, batch, 0)


def kernel(classes):
    cls_flat = classes.reshape(-1)
    zeros_help = jnp.zeros((BATCH_WORDS,), jnp.int32)
    eye = jnp.eye(16, dtype=jnp.int32)
    table = jnp.concatenate([eye, 1 - eye]).reshape(-1)
    flat = _sc_onehot(cls_flat, zeros_help, table)
    return flat.reshape(4096, 20, NUM_CLASSES)


# speedup vs baseline: 2.6075x; 1.5147x over previous
import functools

import jax
import jax.numpy as jnp
from jax import lax
from jax.experimental import pallas as pl
from jax.experimental.pallas import tpu as pltpu
from jax.experimental.pallas import tpu_sc as plsc

NUM_CLASSES = 1000
N_PLANES = 4096
PLANE_ROWS = 20
NW = 32
PLANES_PER_W = N_PLANES // NW
NB = 2
PAIR_PLANES = 2 * NB
PAIR_ROWS = PAIR_PLANES * PLANE_ROWS
PAIRS = PLANES_PER_W // PAIR_PLANES

_mesh = plsc.VectorSubcoreMesh(core_axis_name="c", subcore_axis_name="s")


@functools.partial(
    pl.kernel,
    mesh=_mesh,
    compiler_params=pltpu.CompilerParams(use_tc_tiling_on_sc=True),
    out_type=jax.ShapeDtypeStruct((N_PLANES, PLANE_ROWS, NUM_CLASSES), jnp.int32),
    scratch_types=[
        pltpu.VMEM((PLANES_PER_W * PLANE_ROWS,), jnp.int32),
        pltpu.VMEM((NB, PLANE_ROWS, NUM_CLASSES), jnp.int32),
        pltpu.VMEM((NB, PLANE_ROWS, NUM_CLASSES), jnp.int32),
        pltpu.SemaphoreType.DMA,
        pltpu.SemaphoreType.DMA,
    ],
)
def _sc_onehot(cls_hbm, zeros_hbm, out_hbm, cls_v, buf_a, buf_b, sem_a, sem_b):
    bufs = (buf_a, buf_b)
    wid = lax.axis_index("s") * 2 + lax.axis_index("c")
    plane0 = wid * PLANES_PER_W
    pltpu.sync_copy(cls_hbm.at[pl.ds(plane0 * PLANE_ROWS, PLANES_PER_W * PLANE_ROWS)],
                    cls_v)
    pltpu.sync_copy(zeros_hbm, buf_a)
    pltpu.sync_copy(zeros_hbm, buf_b)

    def scatter_pair(pr, val):
        loc_lanes = lax.iota(jnp.int32, 16)
        for g in range(PAIR_ROWS // 16):
            r0 = pr * PAIR_ROWS + g * 16
            cls16 = cls_v[pl.ds(r0, 16)]
            for k in range(16):
                r = g * 16 + k
                pg, j = divmod(r, PLANE_ROWS)
                buf = bufs[pg // NB]
                c = cls16[k]
                cb = pl.multiple_of(c & ~15, 16)
                d = jnp.minimum(jnp.abs(loc_lanes - (c - cb)), 31)
                vec = (jnp.int32(1) >> d) * val
                buf[pg % NB, j, pl.ds(cb, 16)] = vec

    def start(pr):
        base = plane0 + pr * PAIR_PLANES
        pltpu.async_copy(buf_a, out_hbm.at[pl.ds(base, NB)], sem_a)
        pltpu.async_copy(buf_b, out_hbm.at[pl.ds(base + NB, NB)], sem_b)

    def wait(pr):
        base = plane0 + pr * PAIR_PLANES
        pltpu.make_async_copy(buf_a, out_hbm.at[pl.ds(base, NB)], sem_a).wait()
        pltpu.make_async_copy(buf_b, out_hbm.at[pl.ds(base + NB, NB)],
                              sem_b).wait()

    scatter_pair(jnp.int32(0), jnp.int32(1))
    start(jnp.int32(0))

    def pair(pr, carry):
        wait(pr - 1)
        scatter_pair(pr - 1, jnp.int32(0))
        scatter_pair(pr, jnp.int32(1))
        start(pr)
        return carry

    lax.fori_loop(1, PAIRS, pair, 0)
    wait(jnp.int32(PAIRS - 1))


def kernel(classes):
    cls_flat = classes.reshape(-1)
    zeros_help = jnp.zeros((NB, PLANE_ROWS, NUM_CLASSES), jnp.int32)
    return _sc_onehot(cls_flat, zeros_help)


# speedup vs baseline: 2.6248x; 1.0067x over previous
import functools

import jax
import jax.numpy as jnp
from jax import lax
from jax.experimental import pallas as pl
from jax.experimental.pallas import tpu as pltpu
from jax.experimental.pallas import tpu_sc as plsc

NUM_CLASSES = 1000
N_PLANES = 4096
PLANE_ROWS = 20
NW = 32
PLANES_PER_W = N_PLANES // NW
NB = 4
ROWS_PER_BATCH = NB * PLANE_ROWS
GROUPS = ROWS_PER_BATCH // 16
BATCHES = PLANES_PER_W // NB

_mesh = plsc.VectorSubcoreMesh(core_axis_name="c", subcore_axis_name="s")


@functools.partial(
    pl.kernel,
    mesh=_mesh,
    compiler_params=pltpu.CompilerParams(use_tc_tiling_on_sc=True),
    out_type=jax.ShapeDtypeStruct((N_PLANES, PLANE_ROWS, NUM_CLASSES), jnp.int32),
    scratch_types=[
        pltpu.VMEM((PLANES_PER_W * PLANE_ROWS,), jnp.int32),
        pltpu.VMEM((NB, PLANE_ROWS, NUM_CLASSES), jnp.int32),
    ],
)
def _sc_onehot(cls_hbm, zeros_hbm, out_hbm, cls_v, buf):
    wid = lax.axis_index("s") * 2 + lax.axis_index("c")
    plane0 = wid * PLANES_PER_W
    pltpu.sync_copy(cls_hbm.at[pl.ds(plane0 * PLANE_ROWS, PLANES_PER_W * PLANE_ROWS)],
                    cls_v)
    pltpu.sync_copy(zeros_hbm, buf)

    def scatter_batch(i, val):
        loc_lanes = lax.iota(jnp.int32, 16)
        for g in range(GROUPS):
            r0 = i * ROWS_PER_BATCH + g * 16
            cls16 = cls_v[pl.ds(r0, 16)]
            for k in range(16):
                r = g * 16 + k
                p, j = divmod(r, PLANE_ROWS)
                c = cls16[k]
                cb = pl.multiple_of(c & ~15, 16)
                d = jnp.minimum(jnp.abs(loc_lanes - (c - cb)), 31)
                vec = (jnp.int32(1) >> d) * val
                buf[p, j, pl.ds(cb, 16)] = vec

    def batch(i, carry):
        scatter_batch(i, jnp.int32(1))
        pltpu.sync_copy(buf, out_hbm.at[pl.ds(plane0 + i * NB, NB)])
        scatter_batch(i, jnp.int32(0))
        return carry

    lax.fori_loop(0, BATCHES, batch, 0)


def kernel(classes):
    cls_flat = classes.reshape(-1)
    zeros_help = jnp.zeros((NB, PLANE_ROWS, NUM_CLASSES), jnp.int32)
    return _sc_onehot(cls_flat, zeros_help)
